# baseline (speedup 1.0000x reference)
import jax
import jax.numpy as jnp
from jax import lax
from jax.experimental import pallas as pl
from jax.experimental.pallas import tpu as pltpu
from jax.experimental.pallas import tpu_sc as plsc

DIM = 64
NC, NS = 2, 16
NW = NC * NS
CH = 64
K = 10


def _gather_body(idx_hbm, table_hbm, out_hbm, idx_v, rows_v, gs, os0, os1):
    wid = lax.axis_index("s") * NC + lax.axis_index("c")
    n_ch = idx_hbm.shape[1]
    base = wid * (n_ch * CH)
    oss = (os0, os1)
    pltpu.sync_copy(idx_hbm.at[wid], idx_v)

    def do_block(o, d):
        descs = [
            pltpu.async_copy(
                table_hbm.at[idx_v.at[o * K + j]],
                rows_v.at[d, pl.ds(j * CH, CH)],
                gs,
            )
            for j in range(K)
        ]
        for desc in descs:
            desc.wait()
        pltpu.async_copy(
            rows_v.at[d],
            out_hbm.at[pl.ds(base + o * K * CH, K * CH)],
            oss[d],
        )

    def outer(p, carry):
        for dd in (0, 1):
            @pl.when(p >= 1)
            def _drain():
                pltpu.make_async_copy(
                    out_hbm.at[pl.ds(0, K * CH)], rows_v.at[dd], oss[dd]
                ).wait()
            do_block(2 * p + dd, dd)
        return carry

    n_blk = n_ch // K
    lax.fori_loop(0, n_blk // 2, outer, 0)
    for dd in (0, 1):
        pltpu.make_async_copy(
            out_hbm.at[pl.ds(0, K * CH)], rows_v.at[dd], oss[dd]
        ).wait()


def kernel(words, W_w, W_c):
    B = words.shape[0] * words.shape[1]
    n_ch = B // (NW * CH)
    idx = words.reshape(NW, n_ch, CH)
    out = pl.kernel(
        _gather_body,
        out_type=jax.ShapeDtypeStruct((B, DIM), jnp.float32),
        mesh=plsc.VectorSubcoreMesh(core_axis_name="c", subcore_axis_name="s"),
        scratch_types=[
            pltpu.VMEM((n_ch, CH), jnp.int32),
            pltpu.VMEM((2, K * CH, DIM), jnp.float32),
            pltpu.SemaphoreType.DMA,
            pltpu.SemaphoreType.DMA,
            pltpu.SemaphoreType.DMA,
        ],
        compiler_params=pltpu.CompilerParams(use_tc_tiling_on_sc=False),
    )(idx, W_w)
    return out.reshape(words.shape[0], words.shape[1], DIM)


# speedup vs baseline: 1.0021x; 1.0021x over previous
import jax
import jax.numpy as jnp
from jax import lax
from jax.experimental import pallas as pl
from jax.experimental.pallas import tpu as pltpu
from jax.experimental.pallas import tpu_sc as plsc

DIM = 64
NC, NS = 2, 16
NW = NC * NS
CH = 128
R = 8
IDX0 = 32


def _gather_body(idx_hbm, table_hbm, out_hbm, idx_v, rows_v, gs, os_, isem):
    wid = lax.axis_index("s") * NC + lax.axis_index("c")
    n_ch = idx_hbm.shape[1]
    base = wid * (n_ch * CH)

    pltpu.sync_copy(idx_hbm.at[wid, pl.ds(0, IDX0)], idx_v.at[pl.ds(0, IDX0)])
    pltpu.async_copy(
        idx_hbm.at[wid, pl.ds(IDX0, n_ch - IDX0)],
        idx_v.at[pl.ds(IDX0, n_ch - IDX0)],
        isem,
    )
    for c in range(R):
        pltpu.async_copy(table_hbm.at[idx_v.at[c]], rows_v.at[c], gs)

    dummy = out_hbm.at[pl.ds(0, CH)]

    def outer(p, carry):
        @pl.when(p == IDX0 // R - 1)
        def _wait_idx():
            pltpu.make_async_copy(
                idx_hbm.at[wid, pl.ds(IDX0, n_ch - IDX0)],
                idx_v.at[pl.ds(IDX0, n_ch - IDX0)],
                isem,
            ).wait()

        for jj in range(R):
            j = p * R + jj
            g = j + R - 2
            sf = (jj - 2) % R

            @pl.when(jnp.logical_and(j >= 2, g < n_ch))
            def _refill():
                pltpu.make_async_copy(dummy, rows_v.at[sf], os_).wait()
                pltpu.async_copy(table_hbm.at[idx_v.at[g]], rows_v.at[sf], gs)

            pltpu.make_async_copy(dummy, rows_v.at[jj], gs).wait()
            pltpu.async_copy(
                rows_v.at[jj], out_hbm.at[pl.ds(base + j * CH, CH)], os_
            )
        return carry

    lax.fori_loop(0, n_ch // R, outer, 0)
    for s in range(R):
        pltpu.make_async_copy(dummy, rows_v.at[s], os_).wait()


def kernel(words, W_w, W_c):
    B = words.shape[0] * words.shape[1]
    n_ch = B // (NW * CH)
    idx = words.reshape(NW, n_ch, CH)
    out = pl.kernel(
        _gather_body,
        out_type=jax.ShapeDtypeStruct((B, DIM), jnp.float32),
        mesh=plsc.VectorSubcoreMesh(core_axis_name="c", subcore_axis_name="s"),
        scratch_types=[
            pltpu.VMEM((n_ch, CH), jnp.int32),
            pltpu.VMEM((R, CH, DIM), jnp.float32),
            pltpu.SemaphoreType.DMA,
            pltpu.SemaphoreType.DMA,
            pltpu.SemaphoreType.DMA,
        ],
        compiler_params=pltpu.CompilerParams(
            use_tc_tiling_on_sc=False, disable_bounds_checks=True
        ),
    )(idx, W_w)
    return out.reshape(words.shape[0], words.shape[1], DIM)
